# Initial kernel scaffold; baseline (speedup 1.0000x reference)
import jax
import jax.numpy as jnp
from jax.experimental import pallas as pl


def kernel(feat, edge_index, W0, W1, Wp1, bp1, gamma, beta, Wp2, bp2):
    raise NotImplementedError("write your pallas kernel here")


# speedup vs baseline: 3.2658x; 3.2658x over previous
import functools

import jax
import jax.numpy as jnp
from jax import lax
from jax.experimental import pallas as pl
from jax.experimental.pallas import tpu as pltpu
from jax.experimental.pallas import tpu_sc as plsc

_N = 10000
_D = 128
_E = 320000

_NC = 2
_NS = 16
_NW = _NC * _NS

_K = 128
_BATCHES = 79
_PER_SUB = _BATCHES * _K
_E_PAD = _NW * _PER_SUB
_NPAD = 10240
_RPS = _NPAD // _NS

_ROWBLK = 1000
_NBLKS = _N // _ROWBLK


def _sc_mesh():
    return plsc.VectorSubcoreMesh(core_axis_name="c", subcore_axis_name="s")


def _sc_degree(dst_pad, zeros128, ones128):
    @functools.partial(
        pl.kernel,
        out_type=jax.ShapeDtypeStruct((_NC, _NPAD, _D), jnp.float32),
        mesh=_sc_mesh(),
        scratch_types=[
            pltpu.VMEM((_K,), jnp.int32),
            pltpu.VMEM((_K, _D), jnp.float32),
            pltpu.VMEM_SHARED((_NPAD, _D), jnp.float32),
        ],
    )
    def k(dst_hbm, z_hbm, ones_hbm, out_hbm, dst_v, ones_v, acc_sh):
        cid = lax.axis_index("c")
        sid = lax.axis_index("s")
        pltpu.sync_copy(ones_hbm, ones_v)
        pltpu.sync_copy(z_hbm.at[pl.ds(sid * _RPS, _RPS)],
                        acc_sh.at[pl.ds(sid * _RPS, _RPS)])
        plsc.subcore_barrier()
        eoff = (cid * _NS + sid) * _PER_SUB

        def body(b, carry):
            base = eoff + b * _K
            pltpu.sync_copy(dst_hbm.at[pl.ds(base, _K)], dst_v)
            pltpu.sync_copy(ones_v, acc_sh.at[dst_v], add=True)
            return carry

        lax.fori_loop(0, _BATCHES, body, 0)
        plsc.subcore_barrier()
        pltpu.sync_copy(acc_sh.at[pl.ds(sid * _RPS, _RPS)],
                        out_hbm.at[cid, pl.ds(sid * _RPS, _RPS)])

    return k(dst_pad, zeros128, ones128)


def _sc_segsum(src_pad, dst_pad, g, zeros128):
    @functools.partial(
        pl.kernel,
        out_type=jax.ShapeDtypeStruct((_NC, _NPAD, _D), jnp.float32),
        mesh=_sc_mesh(),
        scratch_types=[
            pltpu.VMEM((_K,), jnp.int32),
            pltpu.VMEM((_K,), jnp.int32),
            pltpu.VMEM((_K, _D), jnp.float32),
            pltpu.VMEM_SHARED((_NPAD, _D), jnp.float32),
            pltpu.SemaphoreType.DMA,
        ],
    )
    def k(src_hbm, dst_hbm, g_hbm, z_hbm, out_hbm,
          src_v, dst_v, rows_v, acc_sh, sem):
        cid = lax.axis_index("c")
        sid = lax.axis_index("s")
        pltpu.sync_copy(z_hbm.at[pl.ds(sid * _RPS, _RPS)],
                        acc_sh.at[pl.ds(sid * _RPS, _RPS)])
        plsc.subcore_barrier()
        eoff = (cid * _NS + sid) * _PER_SUB

        def body(b, carry):
            base = eoff + b * _K
            pltpu.sync_copy(src_hbm.at[pl.ds(base, _K)], src_v)
            pltpu.sync_copy(dst_hbm.at[pl.ds(base, _K)], dst_v)
            pltpu.async_copy(g_hbm.at[src_v], rows_v, sem).wait()
            pltpu.sync_copy(rows_v, acc_sh.at[dst_v], add=True)
            return carry

        lax.fori_loop(0, _BATCHES, body, 0)
        plsc.subcore_barrier()
        pltpu.sync_copy(acc_sh.at[pl.ds(sid * _RPS, _RPS)],
                        out_hbm.at[cid, pl.ds(sid * _RPS, _RPS)])

    return k(src_pad, dst_pad, g, zeros128)


def _tc_prep(deg_acc, feat):
    def body(deg_ref, feat_ref, g_ref, dinv_ref):
        deg = deg_ref[0, :, 0:8] + deg_ref[1, :, 0:8]
        dinv = lax.rsqrt(jnp.maximum(deg, 1.0))
        dinv_ref[...] = dinv
        g_ref[...] = feat_ref[...] * dinv[:, 0:1]

    return pl.pallas_call(
        body,
        grid=(_NBLKS,),
        in_specs=[
            pl.BlockSpec((2, _ROWBLK, _D), lambda i: (0, i, 0)),
            pl.BlockSpec((_ROWBLK, _D), lambda i: (i, 0)),
        ],
        out_specs=[
            pl.BlockSpec((_ROWBLK, _D), lambda i: (i, 0)),
            pl.BlockSpec((_ROWBLK, 8), lambda i: (i, 0)),
        ],
        out_shape=[
            jax.ShapeDtypeStruct((_N, _D), jnp.float32),
            jax.ShapeDtypeStruct((_N, 8), jnp.float32),
        ],
    )(deg_acc, feat)


def _tc_mid(a1_acc, dinv8):
    def body(a1_ref, dinv_ref, g2_ref):
        a1 = a1_ref[0] + a1_ref[1]
        dinv = dinv_ref[:, 0:1]
        g2_ref[...] = a1 * (dinv * dinv)

    return pl.pallas_call(
        body,
        grid=(_NBLKS,),
        in_specs=[
            pl.BlockSpec((2, _ROWBLK, _D), lambda i: (0, i, 0)),
            pl.BlockSpec((_ROWBLK, 8), lambda i: (i, 0)),
        ],
        out_specs=pl.BlockSpec((_ROWBLK, _D), lambda i: (i, 0)),
        out_shape=jax.ShapeDtypeStruct((_N, _D), jnp.float32),
    )(a1_acc, dinv8)


def _tc_head(feat, a1_acc, a2_acc, dinv8, W0, W1, Wp1, bp1):
    def body(feat_ref, a1_ref, a2_ref, dinv_ref, W0_ref, W1_ref, Wp1_ref,
             bp1_ref, t_ref, s1_ref, s2_ref):
        i = pl.program_id(0)
        dinv = dinv_ref[:, 0:1]
        a1 = a1_ref[0] + a1_ref[1]
        a2 = a2_ref[0] + a2_ref[1]
        z = feat_ref[...] - 2.0 * dinv * a1 + dinv * a2
        h = jnp.dot(z, W0_ref[...], preferred_element_type=jnp.float32)
        h = jnp.dot(h, W1_ref[...], preferred_element_type=jnp.float32)
        t = jnp.dot(h, Wp1_ref[...], preferred_element_type=jnp.float32)
        t = t + bp1_ref[...]
        t_ref[...] = t

        @pl.when(i == 0)
        def _():
            s1_ref[...] = jnp.zeros_like(s1_ref)
            s2_ref[...] = jnp.zeros_like(s2_ref)

        s1_ref[...] += jnp.sum(t, axis=0, keepdims=True)
        s2_ref[...] += jnp.sum(t * t, axis=0, keepdims=True)

    nmid = Wp1.shape[1]
    return pl.pallas_call(
        body,
        grid=(_NBLKS,),
        in_specs=[
            pl.BlockSpec((_ROWBLK, _D), lambda i: (i, 0)),
            pl.BlockSpec((2, _ROWBLK, _D), lambda i: (0, i, 0)),
            pl.BlockSpec((2, _ROWBLK, _D), lambda i: (0, i, 0)),
            pl.BlockSpec((_ROWBLK, 8), lambda i: (i, 0)),
            pl.BlockSpec(W0.shape, lambda i: (0, 0)),
            pl.BlockSpec(W1.shape, lambda i: (0, 0)),
            pl.BlockSpec(Wp1.shape, lambda i: (0, 0)),
            pl.BlockSpec((1, nmid), lambda i: (0, 0)),
        ],
        out_specs=[
            pl.BlockSpec((_ROWBLK, nmid), lambda i: (i, 0)),
            pl.BlockSpec((1, nmid), lambda i: (0, 0)),
            pl.BlockSpec((1, nmid), lambda i: (0, 0)),
        ],
        out_shape=[
            jax.ShapeDtypeStruct((_N, nmid), jnp.float32),
            jax.ShapeDtypeStruct((1, nmid), jnp.float32),
            jax.ShapeDtypeStruct((1, nmid), jnp.float32),
        ],
    )(feat, a1_acc, a2_acc, dinv8, W0, W1, Wp1, bp1)


def _tc_bn_out(t, s1, s2, gamma, beta, Wp2, bp2):
    def body(t_ref, s1_ref, s2_ref, g_ref, b_ref, Wp2_ref, bp2_ref, o_ref):
        inv_n = 1.0 / _N
        mean = s1_ref[...] * inv_n
        var = s2_ref[...] * inv_n - mean * mean
        scale = g_ref[...] * lax.rsqrt(var + 1e-5)
        x = (t_ref[...] - mean) * scale + b_ref[...]
        x = jnp.maximum(x, 0.0)
        o_ref[...] = (jnp.dot(x, Wp2_ref[...],
                              preferred_element_type=jnp.float32)
                      + bp2_ref[...])

    nmid = Wp2.shape[0]
    nout = Wp2.shape[1]
    return pl.pallas_call(
        body,
        grid=(_NBLKS,),
        in_specs=[
            pl.BlockSpec((_ROWBLK, nmid), lambda i: (i, 0)),
            pl.BlockSpec((1, nmid), lambda i: (0, 0)),
            pl.BlockSpec((1, nmid), lambda i: (0, 0)),
            pl.BlockSpec((1, nmid), lambda i: (0, 0)),
            pl.BlockSpec((1, nmid), lambda i: (0, 0)),
            pl.BlockSpec(Wp2.shape, lambda i: (0, 0)),
            pl.BlockSpec((1, nout), lambda i: (0, 0)),
        ],
        out_specs=pl.BlockSpec((_ROWBLK, nout), lambda i: (i, 0)),
        out_shape=jax.ShapeDtypeStruct((_N, nout), jnp.float32),
    )(t, s1, s2, gamma, beta, Wp2, bp2)


def kernel(feat, edge_index, W0, W1, Wp1, bp1, gamma, beta, Wp2, bp2):
    src = edge_index[0]
    dst = edge_index[1]
    npad = _E_PAD - _E
    src_pad = jnp.concatenate([src, jnp.zeros((npad,), jnp.int32)])
    dst_pad = jnp.concatenate([dst, jnp.full((npad,), _N, jnp.int32)])
    zeros128 = jnp.zeros((_NPAD, _D), jnp.float32)
    ones128 = jnp.ones((_K, _D), jnp.float32)

    deg_acc = _sc_degree(dst_pad, zeros128, ones128)
    g, dinv8 = _tc_prep(deg_acc, feat)
    a1_acc = _sc_segsum(src_pad, dst_pad, g, zeros128)
    g2 = _tc_mid(a1_acc, dinv8)
    a2_acc = _sc_segsum(src_pad, dst_pad, g2, zeros128)
    t, s1, s2 = _tc_head(feat, a1_acc, a2_acc, dinv8, W0, W1, Wp1,
                         bp1.reshape(1, -1))
    return _tc_bn_out(t, s1, s2, gamma.reshape(1, -1), beta.reshape(1, -1),
                      Wp2, bp2.reshape(1, -1))
